# baseline (speedup 1.0000x reference)
import functools

import jax
import jax.numpy as jnp
from jax import lax
from jax.experimental import pallas as pl
from jax.experimental.pallas import tpu as pltpu
from jax.experimental.pallas import tpu_sc as plsc

N_NODES = 100000
N_ELEM = 128
ROWS = 160
CHUNK = ROWS * N_ELEM
N_CHUNKS = N_NODES // ROWS
NW = 32
N_ITERS = -(-N_CHUNKS // NW)
GPC = ROWS // 16
IDX_COLS = 80
IDX_ROWS = N_ITERS * ROWS // IDX_COLS


def _onehot_body(types_hbm, out_hbm, zbuf, ones_v, types_v, idx_t,
                 sem_t, sem_z, sem_s):
    wid = lax.axis_index("s") * 2 + lax.axis_index("c")
    iota = lax.iota(jnp.int32, 16)
    zeros16 = jnp.zeros((16,), jnp.float32)
    ones16 = jnp.ones((16,), jnp.float32)

    for i in range(N_ITERS):
        c = wid + NW * i
        c_eff = jnp.where(c < N_CHUNKS, c, 0)
        pltpu.make_async_copy(
            types_hbm.at[pl.ds(c_eff * ROWS, ROWS)],
            types_v.at[pl.ds(i * ROWS, ROWS)],
            sem_t,
        ).start()

    def zbody(j, _):
        zbuf[pl.ds(j * 16, 16)] = zeros16
        return 0
    lax.fori_loop(0, CHUNK // 16, zbody, 0)
    for j in range(IDX_COLS // 16):
        ones_v[pl.ds(16 * j, 16)] = ones16

    for i in range(N_ITERS):
        c = wid + NW * i

        @pl.when(c < N_CHUNKS)
        def _():
            pltpu.make_async_copy(
                zbuf, out_hbm.at[pl.ds(c * CHUNK, CHUNK)], sem_z
            ).start()

    for i in range(N_ITERS):
        pltpu.make_async_copy(
            types_hbm.at[pl.ds(0, ROWS)],
            types_v.at[pl.ds(i * ROWS, ROWS)],
            sem_t,
        ).wait()

    idx0 = None
    for i in range(N_ITERS):
        c = wid + NW * i
        valid = c < N_CHUNKS
        for j in range(GPC):
            tv = types_v[pl.ds(i * ROWS + 16 * j, 16)]
            rows = (c * ROWS + 16 * j) + iota
            idx = rows * N_ELEM + tv
            if idx0 is None:
                idx0 = idx
            else:
                idx = jnp.where(valid, idx, idx0)
            g = i * GPC + j
            idx_t[g // 5, pl.ds(16 * (g % 5), 16)] = idx

    for i in range(N_ITERS):
        c = wid + NW * i

        @pl.when(c < N_CHUNKS)
        def _():
            pltpu.make_async_copy(
                zbuf, out_hbm.at[pl.ds(0, CHUNK)], sem_z
            ).wait()

    for r in range(IDX_ROWS):
        pltpu.make_async_copy(
            ones_v, out_hbm.at[idx_t.at[r]], sem_s
        ).start()
    for r in range(IDX_ROWS):
        pltpu.make_async_copy(
            ones_v, out_hbm.at[idx_t.at[0]], sem_s
        ).wait()


@jax.jit
def _onehot_sc(atomic_types):
    mesh = plsc.VectorSubcoreMesh(core_axis_name="c", subcore_axis_name="s")
    f = functools.partial(
        pl.kernel,
        mesh=mesh,
        compiler_params=pltpu.CompilerParams(
            needs_layout_passes=False,
            use_tc_tiling_on_sc=False,
        ),
        out_type=jax.ShapeDtypeStruct((N_NODES * N_ELEM,), jnp.float32),
        scratch_types=[
            pltpu.VMEM((CHUNK,), jnp.float32),
            pltpu.VMEM((IDX_COLS,), jnp.float32),
            pltpu.VMEM((N_ITERS * ROWS,), jnp.int32),
            pltpu.VMEM((IDX_ROWS, IDX_COLS), jnp.int32),
            pltpu.SemaphoreType.DMA,
            pltpu.SemaphoreType.DMA,
            pltpu.SemaphoreType.DMA,
        ],
    )(_onehot_body)
    return f(atomic_types)


def kernel(atomic_types, positions):
    del positions
    return _onehot_sc(atomic_types).reshape(N_NODES, N_ELEM)


# speedup vs baseline: 1.4870x; 1.4870x over previous
import jax
import jax.numpy as jnp
from jax import lax
from jax.experimental import pallas as pl
from jax.experimental.pallas import tpu as pltpu

N_NODES = 100000
N_ELEM = 128
BLK = 2000


def _body(t_ref, o_ref):
    t = t_ref[...]
    iota = lax.broadcasted_iota(jnp.int32, (BLK, N_ELEM), 1)
    o_ref[...] = (iota == t).astype(jnp.float32)


@jax.jit
def _onehot_tc(atomic_types):
    t2 = atomic_types.reshape(N_NODES, 1)
    return pl.pallas_call(
        _body,
        grid=(N_NODES // BLK,),
        in_specs=[pl.BlockSpec((BLK, 1), lambda i: (i, 0))],
        out_specs=pl.BlockSpec((BLK, N_ELEM), lambda i: (i, 0)),
        out_shape=jax.ShapeDtypeStruct((N_NODES, N_ELEM), jnp.float32),
    )(t2)


def kernel(atomic_types, positions):
    del positions
    return _onehot_tc(atomic_types)


# speedup vs baseline: 2.9096x; 1.9566x over previous
import functools

import jax
import jax.numpy as jnp
from jax import lax
from jax.experimental import pallas as pl
from jax.experimental.pallas import tpu as pltpu
from jax.experimental.pallas import tpu_sc as plsc

N_NODES = 100000
N_ELEM = 128
ROWS = 160
CHUNK = ROWS * N_ELEM
N_CHUNKS = N_NODES // ROWS
NW = 32
N_ITERS = -(-N_CHUNKS // NW)
NBUF = 2


def _onehot_body(types_hbm, out_hbm, buf0, buf1, types_v, sem_t, sem0, sem1):
    wid = lax.axis_index("s") * 2 + lax.axis_index("c")
    bufs = (buf0, buf1)
    sems = (sem0, sem1)
    iota = lax.iota(jnp.int32, 16)
    ones = jnp.ones((16,), jnp.float32)
    zeros = jnp.zeros((16,), jnp.float32)

    for i in range(N_ITERS):
        c = wid + NW * i

        @pl.when(c < N_CHUNKS)
        def _():
            pltpu.make_async_copy(
                types_hbm.at[pl.ds(c * ROWS, ROWS)],
                types_v.at[pl.ds(i * ROWS, ROWS)],
                sem_t,
            ).start()

    for buf in bufs:
        def zbody(j, _, buf=buf):
            buf[pl.ds(j * 16, 16)] = zeros
            return 0
        lax.fori_loop(0, CHUNK // 16, zbody, 0)

    for i in range(N_ITERS):
        c = wid + NW * i

        @pl.when(c < N_CHUNKS)
        def _():
            pltpu.make_async_copy(
                types_hbm.at[pl.ds(0, ROWS)],
                types_v.at[pl.ds(i * ROWS, ROWS)],
                sem_t,
            ).wait()

    for i in range(N_ITERS):
        c = wid + NW * i
        b = i % NBUF

        @pl.when(c < N_CHUNKS)
        def _():
            if i >= NBUF:
                c_old = wid + NW * (i - NBUF)
                pltpu.make_async_copy(
                    bufs[b], out_hbm.at[pl.ds(c_old * CHUNK, CHUNK)], sems[b]
                ).wait()
                for j in range(ROWS // 16):
                    tv = types_v[pl.ds((i - NBUF) * ROWS + 16 * j, 16)]
                    idx = (16 * j + iota) * N_ELEM + tv
                    plsc.store_scatter(bufs[b], [idx], zeros)
            for j in range(ROWS // 16):
                tv = types_v[pl.ds(i * ROWS + 16 * j, 16)]
                idx = (16 * j + iota) * N_ELEM + tv
                plsc.store_scatter(bufs[b], [idx], ones)
            pltpu.make_async_copy(
                bufs[b], out_hbm.at[pl.ds(c * CHUNK, CHUNK)], sems[b]
            ).start()

    for b in range(NBUF):
        pltpu.make_async_copy(
            bufs[b], out_hbm.at[pl.ds(0, CHUNK)], sems[b]
        ).wait()


@jax.jit
def _onehot_sc(atomic_types):
    mesh = plsc.VectorSubcoreMesh(core_axis_name="c", subcore_axis_name="s")
    f = functools.partial(
        pl.kernel,
        mesh=mesh,
        compiler_params=pltpu.CompilerParams(
            needs_layout_passes=False,
            use_tc_tiling_on_sc=False,
        ),
        out_type=jax.ShapeDtypeStruct((N_NODES * N_ELEM,), jnp.float32),
        scratch_types=[
            pltpu.VMEM((CHUNK,), jnp.float32),
            pltpu.VMEM((CHUNK,), jnp.float32),
            pltpu.VMEM((N_ITERS * ROWS,), jnp.int32),
            pltpu.SemaphoreType.DMA,
            pltpu.SemaphoreType.DMA,
            pltpu.SemaphoreType.DMA,
        ],
    )(_onehot_body)
    return f(atomic_types)


def kernel(atomic_types, positions):
    del positions
    return _onehot_sc(atomic_types).reshape(N_NODES, N_ELEM)
